# baseline (speedup 1.0000x reference)
import jax
import jax.numpy as jnp
from jax.experimental import pallas as pl

N = 10000
G = 512
K = 30


def _conv1d(x, w, b, stride):
    out = jax.lax.conv_general_dilated(x, w, (stride,), 'VALID',
                                       dimension_numbers=('NCH', 'OIH', 'NCH'))
    return out + b[None, :, None]


def _final_sigmoid_kernel(z_ref, o_ref):
    o_ref[...] = jax.nn.sigmoid(z_ref[...])


def kernel(x, W1, b1, W2, b2, W3, b3, W4, b4, conv1_w, conv1_b, conv2_w, conv2_b,
           lin1_w, lin1_b, lin2_w, lin2_b, edge_index, batch):
    loops = jnp.arange(N, dtype=edge_index.dtype)
    src = jnp.concatenate([edge_index[0], loops])
    dst = jnp.concatenate([edge_index[1], loops])
    deg = jax.ops.segment_sum(jnp.ones_like(src, dtype=jnp.float32), dst, num_segments=N)
    dinv = jax.lax.rsqrt(jnp.maximum(deg, 1.0))
    norm = dinv[src] * dinv[dst]

    def gcn(h, W, b):
        hw = h @ W
        msg = hw[src] * norm[:, None]
        return jax.ops.segment_sum(msg, dst, num_segments=N) + b

    h1 = jnp.tanh(gcn(x, W1, b1))
    h2 = jnp.tanh(gcn(h1, W2, b2))
    h3 = jnp.tanh(gcn(h2, W3, b3))
    h4 = jnp.tanh(gcn(h3, W4, b4))
    h = jnp.concatenate([h1, h2, h3, h4], axis=-1)
    D = h.shape[-1]

    sort_key = batch.astype(jnp.float32) * 4.0 - h[:, -1]
    order = jnp.argsort(sort_key)
    hs = h[order]
    bs = batch[order]
    counts = jnp.bincount(batch, length=G)
    starts = jnp.cumsum(counts) - counts
    pos = jnp.arange(N) - starts[bs]
    valid = pos < K
    gi = jnp.where(valid, bs, G)
    pi = jnp.where(valid, pos, 0)
    pooled = jnp.zeros((G + 1, K, D), dtype=h.dtype).at[gi, pi].set(hs)[:G]

    z = pooled.reshape(G, 1, K * D)
    z = jax.nn.relu(_conv1d(z, conv1_w, conv1_b, 97))
    L = z.shape[-1]
    z = z[..., :(L // 2) * 2].reshape(G, z.shape[1], L // 2, 2).max(-1)
    z = jax.nn.relu(_conv1d(z, conv2_w, conv2_b, 1))
    z = z.reshape(G, -1)
    z = jax.nn.relu(z @ lin1_w + lin1_b)
    z = z @ lin2_w + lin2_b
    return pl.pallas_call(
        _final_sigmoid_kernel,
        out_shape=jax.ShapeDtypeStruct(z.shape, z.dtype),
    )(z)


# speedup vs baseline: 7.1288x; 7.1288x over previous
import functools

import jax
import jax.numpy as jnp
from jax import lax
from jax.experimental import pallas as pl
from jax.experimental.pallas import tpu as pltpu
from jax.experimental.pallas import tpu_sc as plsc

N = 10000
E = 320000
G = 512
K = 30

NC = 2
NS = 16
NW = NC * NS
EPW = E // NW
CHUNK = 80
NCHUNK = EPW // CHUNK
NBUF = 5
NPAD = 10240
RPT = NPAD // NS

_mesh = plsc.VectorSubcoreMesh(core_axis_name="c", subcore_axis_name="s")


@functools.partial(
    pl.kernel,
    out_type=jax.ShapeDtypeStruct((NC * NPAD, 8), jnp.float32),
    mesh=_mesh,
    scratch_types=[
        pltpu.VMEM((NCHUNK, CHUNK), jnp.int32),
        pltpu.VMEM((CHUNK, 8), jnp.float32),
        pltpu.VMEM_SHARED((NPAD, 8), jnp.float32),
        [pltpu.SemaphoreType.DMA] * NBUF,
    ],
    compiler_params=pltpu.CompilerParams(use_tc_tiling_on_sc=False),
)
def _deg_kernel(dst_hbm, ones_hbm, zeros_hbm, out_hbm, dst_v, ones_v, acc_sh, sem):
    c = lax.axis_index("c")
    s = lax.axis_index("s")
    wid = c * NS + s
    r0 = s * RPT

    pltpu.sync_copy(ones_hbm, ones_v)
    pltpu.sync_copy(zeros_hbm.at[pl.ds(r0, RPT)], acc_sh.at[pl.ds(r0, RPT)])
    pltpu.sync_copy(dst_hbm.at[wid], dst_v)
    plsc.subcore_barrier()

    @pl.loop(0, NCHUNK, step=NBUF)
    def _(j0):
        for b in range(NBUF):
            pltpu.async_copy(ones_v, acc_sh.at[dst_v.at[j0 + b]], sem[b],
                             add=True)
        for b in range(NBUF):
            pltpu.make_async_copy(ones_v, acc_sh.at[dst_v.at[j0]],
                                  sem[b]).wait()

    plsc.subcore_barrier()
    pltpu.sync_copy(acc_sh.at[pl.ds(r0, RPT)],
                    out_hbm.at[pl.ds(c * NPAD + r0, RPT)])


def _make_mp(Hc):
    @functools.partial(
        pl.kernel,
        out_type=jax.ShapeDtypeStruct((NC * NPAD, Hc), jnp.float32),
        mesh=_mesh,
        scratch_types=[
            pltpu.VMEM((NCHUNK, CHUNK), jnp.int32),
            pltpu.VMEM((NCHUNK, CHUNK), jnp.int32),
            [pltpu.VMEM((CHUNK, Hc), jnp.float32)] * NBUF,
            pltpu.VMEM_SHARED((NPAD, Hc), jnp.float32),
            [pltpu.SemaphoreType.DMA] * NBUF,
        ],
        compiler_params=pltpu.CompilerParams(use_tc_tiling_on_sc=False),
    )
    def _mp(hw_hbm, src_hbm, dst_hbm, zeros_hbm, out_hbm,
            src_v, dst_v, msg, acc_sh, gsem):
        c = lax.axis_index("c")
        s = lax.axis_index("s")
        wid = c * NS + s
        r0 = s * RPT

        pltpu.sync_copy(zeros_hbm.at[pl.ds(r0, RPT)], acc_sh.at[pl.ds(r0, RPT)])
        pltpu.sync_copy(src_hbm.at[wid], src_v)
        pltpu.sync_copy(dst_hbm.at[wid], dst_v)
        plsc.subcore_barrier()

        for b in range(NBUF - 1):
            pltpu.async_copy(hw_hbm.at[src_v.at[b]], msg[b], gsem[b])

        @pl.loop(0, NCHUNK, step=NBUF)
        def _(j0):
            for b in range(NBUF):
                j = j0 + b
                pltpu.make_async_copy(hw_hbm.at[src_v.at[j]], msg[b],
                                      gsem[b]).wait()
                pltpu.sync_copy(msg[b], acc_sh.at[dst_v.at[j]], add=True)
                jp = j + NBUF - 1
                pb = (b - 1) % NBUF

                @pl.when(jp < NCHUNK)
                def _():
                    pltpu.async_copy(hw_hbm.at[src_v.at[jp]], msg[pb], gsem[pb])

        plsc.subcore_barrier()
        pltpu.sync_copy(acc_sh.at[pl.ds(r0, RPT)],
                        out_hbm.at[pl.ds(c * NPAD + r0, RPT)])

    return _mp


_mp32 = _make_mp(32)
_mp16 = _make_mp(16)


GPT = G // NW


@functools.partial(
    pl.kernel,
    out_type=(jax.ShapeDtypeStruct((G * 15, 128), jnp.float32),
              jax.ShapeDtypeStruct((G * 15, 128), jnp.float32)),
    mesh=_mesh,
    scratch_types=[
        pltpu.VMEM((NPAD,), jnp.float32),
        pltpu.VMEM((G + 32,), jnp.int32),
        pltpu.VMEM((2, 128), jnp.int32),
        pltpu.VMEM((2, 128), jnp.int32),
        pltpu.VMEM((256, 128), jnp.float32),
        pltpu.VMEM((256, 128), jnp.float32),
        pltpu.SemaphoreType.DMA,
    ],
    compiler_params=pltpu.CompilerParams(use_tc_tiling_on_sc=False,
                                         needs_layout_passes=False),
)
def _sel_kernel(h4_hbm, starts_hbm, hcat_hbm, pe_hbm, po_hbm,
                val_v, st_v, selE, selO, rowsE, rowsO, sem):
    c = lax.axis_index("c")
    s = lax.axis_index("s")
    wid = c * NS + s
    g0 = wid * GPT

    pltpu.sync_copy(h4_hbm, val_v)
    pltpu.sync_copy(starts_hbm, st_v)
    nsplat = jnp.full((16,), N, jnp.int32)
    for r in range(2):
        for i in range(8):
            selE[r, pl.ds(i * 16, 16)] = nsplat
            selO[r, pl.ds(i * 16, 16)] = nsplat

    lane = lax.iota(jnp.int32, 16)
    sA = st_v[pl.ds(g0, 16)]
    sC = st_v[pl.ds(g0 + 16, 16)]
    sts = [sA[i] for i in range(16)] + [sC[0]]
    for g_local in range(GPT):
        s0 = sts[g_local]
        s1 = sts[g_local + 1]
        a0 = (s0 >> 4) << 4
        nch = (s1 - a0 + 15) >> 4

        @pl.loop(0, nch)
        def _(ic):
            base = a0 + ic * 16
            vi = val_v[pl.ds(base, 16)]
            gidx = base + lane
            ivalid = (gidx >= s0) & (gidx < s1)

            def jstep(jc, rank):
                jbase = a0 + jc * 16
                vj = val_v[pl.ds(jbase, 16)]
                for l in range(16):
                    bval = vj[l]
                    bidxv = jnp.broadcast_to(jbase + l, (16,))
                    jvv = (bidxv >= s0) & (bidxv < s1)
                    beats = (bval > vi) | ((bval == vi) & (bidxv < gidx))
                    rank = rank + jnp.where(jvv & beats, 1, 0)
                return rank

            rank = pl.loop(0, nch, init_carry=jnp.zeros((16,), jnp.int32))(jstep)

            sel_mask = ivalid & (rank < K)
            te = rank >> 1
            slot = g_local * 15 + te
            row = slot >> 7
            col = slot & 127
            par_odd = (rank & 1) == 1
            plsc.store_scatter(selE, [row, col], gidx, mask=sel_mask & (~par_odd))
            plsc.store_scatter(selO, [row, col], gidx, mask=sel_mask & par_odd)

    for r in range(2):
        pltpu.async_copy(hcat_hbm.at[selE.at[r]],
                         rowsE.at[pl.ds(r * 128, 128)], sem)
        pltpu.make_async_copy(hcat_hbm.at[selE.at[r]],
                              rowsE.at[pl.ds(r * 128, 128)], sem).wait()
        pltpu.async_copy(hcat_hbm.at[selO.at[r]],
                         rowsO.at[pl.ds(r * 128, 128)], sem)
        pltpu.make_async_copy(hcat_hbm.at[selO.at[r]],
                              rowsO.at[pl.ds(r * 128, 128)], sem).wait()

    pltpu.sync_copy(rowsE.at[pl.ds(0, GPT * 15)],
                    pe_hbm.at[pl.ds(wid * GPT * 15, GPT * 15)])
    pltpu.sync_copy(rowsO.at[pl.ds(0, GPT * 15)],
                    po_hbm.at[pl.ds(wid * GPT * 15, GPT * 15)])


def _pre_body(x_ref, w_ref, degp_ref, hwp_ref, dinv_ref):
    deg = 1.0 + degp_ref[0, :, 0] + degp_ref[1, :, 0]
    dinv = lax.rsqrt(deg)[:, None]
    hw = jnp.dot(x_ref[...], w_ref[...], preferred_element_type=jnp.float32)
    hwp_ref[...] = hw * dinv
    dinv_ref[...] = dinv


def _tc_pre(xp, W1, deg_p):
    return pl.pallas_call(
        _pre_body,
        out_shape=(jax.ShapeDtypeStruct((NPAD, 32), jnp.float32),
                   jax.ShapeDtypeStruct((NPAD, 1), jnp.float32)),
    )(xp, W1, deg_p)


def _mid_body(acc_ref, hw_ref, dinv_ref, w_ref, b_ref, h_ref, hwp_ref):
    agg = acc_ref[0] + acc_ref[1] + hw_ref[...]
    h = jnp.tanh(dinv_ref[...] * agg + b_ref[...])
    h_ref[...] = h
    hwp_ref[...] = jnp.dot(h, w_ref[...],
                           preferred_element_type=jnp.float32) * dinv_ref[...]


def _tc_mid(acc, hwp, dinv, Wn, b):
    return pl.pallas_call(
        _mid_body,
        out_shape=(jax.ShapeDtypeStruct((NPAD, 32), jnp.float32),
                   jax.ShapeDtypeStruct((NPAD, 32), jnp.float32)),
    )(acc, hwp, dinv, Wn, b.reshape(1, 32))


def _mid4_body(acc_ref, hw_ref, dinv_ref, w_ref, b_ref, h_ref, hwp_ref):
    agg = acc_ref[0] + acc_ref[1] + hw_ref[...]
    h = jnp.tanh(dinv_ref[...] * agg + b_ref[...])
    h_ref[...] = h
    hw4 = jnp.dot(h, w_ref[...], preferred_element_type=jnp.float32) * dinv_ref[...]
    hwp_ref[...] = jnp.pad(hw4, ((0, 0), (0, 15)))


def _tc_mid4(acc, hwp, dinv, W4, b):
    return pl.pallas_call(
        _mid4_body,
        out_shape=(jax.ShapeDtypeStruct((NPAD, 32), jnp.float32),
                   jax.ShapeDtypeStruct((NPAD, 16), jnp.float32)),
    )(acc, hwp, dinv, W4, b.reshape(1, 32))


def _post_body(acc_ref, hw_ref, dinv_ref, b_ref, h4_ref):
    agg = acc_ref[0, :, :1] + acc_ref[1, :, :1] + hw_ref[:, :1]
    h4_ref[...] = jnp.tanh(dinv_ref[...] * agg + b_ref[...])


def _tc_post(acc4, hwp4, dinv, b4):
    return pl.pallas_call(
        _post_body,
        out_shape=jax.ShapeDtypeStruct((NPAD, 1), jnp.float32),
    )(acc4, hwp4, dinv, b4.reshape(1, 1))


def _headA_body(pe_ref, po_ref, wc_ref, bc_ref, zp_ref):
    ze = jnp.dot(pe_ref[...], wc_ref[...], preferred_element_type=jnp.float32)
    zo = jnp.dot(po_ref[...], wc_ref[...], preferred_element_type=jnp.float32)
    zp_ref[...] = jnp.maximum(jnp.maximum(ze, zo) + bc_ref[...], 0.0)


def _tc_headA(pe, po, wc, bc):
    return pl.pallas_call(
        _headA_body,
        out_shape=jax.ShapeDtypeStruct((G * 15, 16), jnp.float32),
    )(pe, po, wc, bc.reshape(1, 16))


def _headB_body(zp_ref, w2_ref, b2_ref, l1_ref, lb1_ref, l2_ref, lb2_ref, out_ref):
    z = jnp.maximum(
        jnp.dot(zp_ref[...], w2_ref[...], preferred_element_type=jnp.float32)
        + b2_ref[...], 0.0)
    z = jnp.maximum(
        jnp.dot(z, l1_ref[...], preferred_element_type=jnp.float32)
        + lb1_ref[...], 0.0)
    zo = jnp.dot(z, l2_ref[...], preferred_element_type=jnp.float32) + lb2_ref[...]
    out_ref[...] = jax.nn.sigmoid(zo)


def _tc_headB(zp, W2big, b2big, lin1_w, lin1_b, lin2_w, lin2_b):
    return pl.pallas_call(
        _headB_body,
        out_shape=jax.ShapeDtypeStruct((G, 1), jnp.float32),
    )(zp, W2big, b2big, lin1_w, lin1_b.reshape(1, 128), lin2_w,
      lin2_b.reshape(1, 1))


def kernel(x, W1, b1, W2, b2, W3, b3, W4, b4, conv1_w, conv1_b, conv2_w, conv2_b,
           lin1_w, lin1_b, lin2_w, lin2_b, edge_index, batch):
    src_r = edge_index[0].reshape(NW, NCHUNK, CHUNK)
    dst_r = edge_index[1].reshape(NW, NCHUNK, CHUNK)
    zeros32 = jnp.zeros((NPAD, 32), jnp.float32)
    zeros8 = jnp.zeros((NPAD, 8), jnp.float32)
    zeros16 = jnp.zeros((NPAD, 16), jnp.float32)
    ones_c8 = jnp.ones((CHUNK, 8), jnp.float32)

    deg_p = _deg_kernel(dst_r, ones_c8, zeros8).reshape(NC, NPAD, 8)
    xp = jnp.pad(x, ((0, NPAD - N), (0, 0)))

    hwp1, dinv = _tc_pre(xp, W1, deg_p)
    acc1 = _mp32(hwp1, src_r, dst_r, zeros32).reshape(NC, NPAD, 32)
    h1, hwp2 = _tc_mid(acc1, hwp1, dinv, W2, b1)
    acc2 = _mp32(hwp2, src_r, dst_r, zeros32).reshape(NC, NPAD, 32)
    h2, hwp3 = _tc_mid(acc2, hwp2, dinv, W3, b2)
    acc3 = _mp32(hwp3, src_r, dst_r, zeros32).reshape(NC, NPAD, 32)
    h3, hwp4 = _tc_mid4(acc3, hwp3, dinv, W4, b3)
    acc4 = _mp16(hwp4, src_r, dst_r, zeros16).reshape(NC, NPAD, 16)
    h4 = _tc_post(acc4, hwp4, dinv, b4)

    counts = jnp.bincount(batch, length=G).astype(jnp.int32)
    starts = jnp.cumsum(counts) - counts
    starts_p = jnp.pad(jnp.concatenate(
        [starts, jnp.array([N], jnp.int32)]), (0, 31))
    hcat = jnp.concatenate([h1, h2, h3, h4], axis=-1)
    hcat = jnp.pad(hcat, ((0, 0), (0, 31))).at[N].set(0.0)
    pe, po = _sel_kernel(h4.reshape(NPAD), starts_p, hcat)

    wc1 = jnp.pad(conv1_w[:, 0, :].T, ((0, 128 - 97), (0, 0)))
    w2big = jnp.zeros((240, 352), jnp.float32)
    blk = conv2_w.transpose(2, 1, 0).reshape(80, 32)
    for t2 in range(11):
        w2big = w2big.at[t2 * 16:t2 * 16 + 80, t2::11].set(blk)
    b2big = jnp.repeat(conv2_b, 11).reshape(1, 352)

    zp = _tc_headA(pe, po, wc1, conv1_b)
    zpf = zp.reshape(G, 240)
    return _tc_headB(zpf, w2big, b2big, lin1_w, lin1_b, lin2_w, lin2_b)


# speedup vs baseline: 21.9040x; 3.0726x over previous
import functools

import jax
import jax.numpy as jnp
from jax import lax
from jax.experimental import pallas as pl
from jax.experimental.pallas import tpu as pltpu
from jax.experimental.pallas import tpu_sc as plsc

N = 10000
E = 320000
G = 512
K = 30

NC = 2
NS = 16
NW = NC * NS
EPW = E // NW
CHUNK = 80
NCHUNK = EPW // CHUNK
NBUF = 5
NPAD = 10240
RPT = NPAD // NS

_mesh = plsc.VectorSubcoreMesh(core_axis_name="c", subcore_axis_name="s")


@functools.partial(
    pl.kernel,
    out_type=jax.ShapeDtypeStruct((NC * NPAD, 8), jnp.float32),
    mesh=_mesh,
    scratch_types=[
        pltpu.VMEM((NCHUNK, CHUNK), jnp.int32),
        pltpu.VMEM((CHUNK, 8), jnp.float32),
        pltpu.VMEM_SHARED((NPAD, 8), jnp.float32),
        [pltpu.SemaphoreType.DMA] * NBUF,
    ],
    compiler_params=pltpu.CompilerParams(use_tc_tiling_on_sc=False),
)
def _deg_kernel(dst_hbm, ones_hbm, zeros_hbm, out_hbm, dst_v, ones_v, acc_sh, sem):
    c = lax.axis_index("c")
    s = lax.axis_index("s")
    wid = c * NS + s
    r0 = s * RPT

    pltpu.sync_copy(ones_hbm, ones_v)
    pltpu.sync_copy(zeros_hbm.at[pl.ds(r0, RPT)], acc_sh.at[pl.ds(r0, RPT)])
    pltpu.sync_copy(dst_hbm.at[wid], dst_v)
    plsc.subcore_barrier()

    @pl.loop(0, NCHUNK, step=NBUF)
    def _(j0):
        for b in range(NBUF):
            pltpu.async_copy(ones_v, acc_sh.at[dst_v.at[j0 + b]], sem[b],
                             add=True)
        for b in range(NBUF):
            pltpu.make_async_copy(ones_v, acc_sh.at[dst_v.at[j0]],
                                  sem[b]).wait()

    plsc.subcore_barrier()
    pltpu.sync_copy(acc_sh.at[pl.ds(r0, RPT)],
                    out_hbm.at[pl.ds(c * NPAD + r0, RPT)])


def _make_mp(Hc):
    @functools.partial(
        pl.kernel,
        out_type=jax.ShapeDtypeStruct((NC * NPAD, Hc), jnp.float32),
        mesh=_mesh,
        scratch_types=[
            pltpu.VMEM((NCHUNK, CHUNK), jnp.int32),
            pltpu.VMEM((NCHUNK, CHUNK), jnp.int32),
            [pltpu.VMEM((CHUNK, Hc), jnp.float32)] * NBUF,
            pltpu.VMEM_SHARED((NPAD, Hc), jnp.float32),
            [pltpu.SemaphoreType.DMA] * NBUF,
        ],
        compiler_params=pltpu.CompilerParams(use_tc_tiling_on_sc=False),
    )
    def _mp(hw_hbm, src_hbm, dst_hbm, zeros_hbm, out_hbm,
            src_v, dst_v, msg, acc_sh, gsem):
        c = lax.axis_index("c")
        s = lax.axis_index("s")
        wid = c * NS + s
        r0 = s * RPT

        pltpu.sync_copy(zeros_hbm.at[pl.ds(r0, RPT)], acc_sh.at[pl.ds(r0, RPT)])
        pltpu.sync_copy(src_hbm.at[wid], src_v)
        pltpu.sync_copy(dst_hbm.at[wid], dst_v)
        plsc.subcore_barrier()

        for b in range(NBUF - 1):
            pltpu.async_copy(hw_hbm.at[src_v.at[b]], msg[b], gsem[b])

        @pl.loop(0, NCHUNK, step=NBUF)
        def _(j0):
            for b in range(NBUF):
                j = j0 + b
                pltpu.make_async_copy(hw_hbm.at[src_v.at[j]], msg[b],
                                      gsem[b]).wait()
                pltpu.sync_copy(msg[b], acc_sh.at[dst_v.at[j]], add=True)
                jp = j + NBUF - 1
                pb = (b - 1) % NBUF

                @pl.when(jp < NCHUNK)
                def _():
                    pltpu.async_copy(hw_hbm.at[src_v.at[jp]], msg[pb], gsem[pb])

        plsc.subcore_barrier()
        pltpu.sync_copy(acc_sh.at[pl.ds(r0, RPT)],
                        out_hbm.at[pl.ds(c * NPAD + r0, RPT)])

    return _mp


_mp32 = _make_mp(32)
_mp16 = _make_mp(16)


GPT = G // NW


@functools.partial(
    pl.kernel,
    out_type=(jax.ShapeDtypeStruct((G * 15, 128), jnp.float32),
              jax.ShapeDtypeStruct((G * 15, 128), jnp.float32)),
    mesh=_mesh,
    scratch_types=[
        pltpu.VMEM((NPAD,), jnp.float32),
        pltpu.VMEM((G + 32,), jnp.int32),
        pltpu.VMEM((2, 128), jnp.int32),
        pltpu.VMEM((2, 128), jnp.int32),
        pltpu.VMEM((256, 128), jnp.float32),
        pltpu.VMEM((256, 128), jnp.float32),
        pltpu.SemaphoreType.DMA,
    ],
    compiler_params=pltpu.CompilerParams(use_tc_tiling_on_sc=False,
                                         needs_layout_passes=False),
)
def _sel_kernel(h4_hbm, starts_hbm, hcat_hbm, pe_hbm, po_hbm,
                val_v, st_v, selE, selO, rowsE, rowsO, sem):
    c = lax.axis_index("c")
    s = lax.axis_index("s")
    wid = c * NS + s
    g0 = wid * GPT

    pltpu.sync_copy(h4_hbm, val_v)
    pltpu.sync_copy(starts_hbm, st_v)
    nsplat = jnp.full((16,), N, jnp.int32)
    for r in range(2):
        for i in range(8):
            selE[r, pl.ds(i * 16, 16)] = nsplat
            selO[r, pl.ds(i * 16, 16)] = nsplat

    lane = lax.iota(jnp.int32, 16)
    sA = st_v[pl.ds(g0, 16)]
    sC = st_v[pl.ds(g0 + 16, 16)]
    sts = [sA[i] for i in range(16)] + [sC[0]]
    for g_local in range(GPT):
        s0 = sts[g_local]
        s1 = sts[g_local + 1]
        a0 = (s0 >> 4) << 4
        nch = (s1 - a0 + 15) >> 4

        @pl.loop(0, nch)
        def _(ic):
            base = a0 + ic * 16
            vi = val_v[pl.ds(base, 16)]
            gidx = base + lane
            ivalid = (gidx >= s0) & (gidx < s1)

            def jstep(jc, rank):
                jbase = a0 + jc * 16
                vj = val_v[pl.ds(jbase, 16)]
                for r in range(16):
                    idxr = (lane + r) & 15
                    vjr = vj[idxr]
                    bidxv = jbase + idxr
                    jvv = (bidxv >= s0) & (bidxv < s1)
                    beats = (vjr > vi) | ((vjr == vi) & (bidxv < gidx))
                    rank = rank + jnp.where(jvv & beats, 1, 0)
                return rank

            rank = pl.loop(0, nch, init_carry=jnp.zeros((16,), jnp.int32))(jstep)

            sel_mask = ivalid & (rank < K)
            te = rank >> 1
            slot = g_local * 15 + te
            row = slot >> 7
            col = slot & 127
            par_odd = (rank & 1) == 1
            plsc.store_scatter(selE, [row, col], gidx, mask=sel_mask & (~par_odd))
            plsc.store_scatter(selO, [row, col], gidx, mask=sel_mask & par_odd)

    for r in range(2):
        pltpu.async_copy(hcat_hbm.at[selE.at[r]],
                         rowsE.at[pl.ds(r * 128, 128)], sem)
        pltpu.make_async_copy(hcat_hbm.at[selE.at[r]],
                              rowsE.at[pl.ds(r * 128, 128)], sem).wait()
        pltpu.async_copy(hcat_hbm.at[selO.at[r]],
                         rowsO.at[pl.ds(r * 128, 128)], sem)
        pltpu.make_async_copy(hcat_hbm.at[selO.at[r]],
                              rowsO.at[pl.ds(r * 128, 128)], sem).wait()

    pltpu.sync_copy(rowsE.at[pl.ds(0, GPT * 15)],
                    pe_hbm.at[pl.ds(wid * GPT * 15, GPT * 15)])
    pltpu.sync_copy(rowsO.at[pl.ds(0, GPT * 15)],
                    po_hbm.at[pl.ds(wid * GPT * 15, GPT * 15)])


def _pre_body(x_ref, w_ref, degp_ref, hwp_ref, dinv_ref):
    deg = 1.0 + degp_ref[0, :, 0] + degp_ref[1, :, 0]
    dinv = lax.rsqrt(deg)[:, None]
    hw = jnp.dot(x_ref[...], w_ref[...], preferred_element_type=jnp.float32)
    hwp_ref[...] = hw * dinv
    dinv_ref[...] = dinv


def _tc_pre(xp, W1, deg_p):
    return pl.pallas_call(
        _pre_body,
        out_shape=(jax.ShapeDtypeStruct((NPAD, 32), jnp.float32),
                   jax.ShapeDtypeStruct((NPAD, 1), jnp.float32)),
    )(xp, W1, deg_p)


def _mid_body(acc_ref, hw_ref, dinv_ref, w_ref, b_ref, h_ref, hwp_ref):
    agg = acc_ref[0] + acc_ref[1] + hw_ref[...]
    h = jnp.tanh(dinv_ref[...] * agg + b_ref[...])
    h_ref[...] = h
    hwp_ref[...] = jnp.dot(h, w_ref[...],
                           preferred_element_type=jnp.float32) * dinv_ref[...]


def _tc_mid(acc, hwp, dinv, Wn, b):
    return pl.pallas_call(
        _mid_body,
        out_shape=(jax.ShapeDtypeStruct((NPAD, 32), jnp.float32),
                   jax.ShapeDtypeStruct((NPAD, 32), jnp.float32)),
    )(acc, hwp, dinv, Wn, b.reshape(1, 32))


def _mid4_body(acc_ref, hw_ref, dinv_ref, w_ref, b_ref, h_ref, hwp_ref):
    agg = acc_ref[0] + acc_ref[1] + hw_ref[...]
    h = jnp.tanh(dinv_ref[...] * agg + b_ref[...])
    h_ref[...] = h
    hw4 = jnp.dot(h, w_ref[...], preferred_element_type=jnp.float32) * dinv_ref[...]
    hwp_ref[...] = jnp.pad(hw4, ((0, 0), (0, 15)))


def _tc_mid4(acc, hwp, dinv, W4, b):
    return pl.pallas_call(
        _mid4_body,
        out_shape=(jax.ShapeDtypeStruct((NPAD, 32), jnp.float32),
                   jax.ShapeDtypeStruct((NPAD, 16), jnp.float32)),
    )(acc, hwp, dinv, W4, b.reshape(1, 32))


def _post_body(acc_ref, hw_ref, dinv_ref, b_ref, h4_ref):
    agg = acc_ref[0, :, :1] + acc_ref[1, :, :1] + hw_ref[:, :1]
    h4_ref[...] = jnp.tanh(dinv_ref[...] * agg + b_ref[...])


def _tc_post(acc4, hwp4, dinv, b4):
    return pl.pallas_call(
        _post_body,
        out_shape=jax.ShapeDtypeStruct((NPAD, 1), jnp.float32),
    )(acc4, hwp4, dinv, b4.reshape(1, 1))


def _headA_body(pe_ref, po_ref, wc_ref, bc_ref, zp_ref):
    ze = jnp.dot(pe_ref[...], wc_ref[...], preferred_element_type=jnp.float32)
    zo = jnp.dot(po_ref[...], wc_ref[...], preferred_element_type=jnp.float32)
    zp_ref[...] = jnp.maximum(jnp.maximum(ze, zo) + bc_ref[...], 0.0)


def _tc_headA(pe, po, wc, bc):
    return pl.pallas_call(
        _headA_body,
        out_shape=jax.ShapeDtypeStruct((G * 15, 16), jnp.float32),
    )(pe, po, wc, bc.reshape(1, 16))


def _headB_body(zp_ref, w2_ref, b2_ref, l1_ref, lb1_ref, l2_ref, lb2_ref, out_ref):
    z = jnp.maximum(
        jnp.dot(zp_ref[...], w2_ref[...], preferred_element_type=jnp.float32)
        + b2_ref[...], 0.0)
    z = jnp.maximum(
        jnp.dot(z, l1_ref[...], preferred_element_type=jnp.float32)
        + lb1_ref[...], 0.0)
    zo = jnp.dot(z, l2_ref[...], preferred_element_type=jnp.float32) + lb2_ref[...]
    out_ref[...] = jax.nn.sigmoid(zo)


def _tc_headB(zp, W2big, b2big, lin1_w, lin1_b, lin2_w, lin2_b):
    return pl.pallas_call(
        _headB_body,
        out_shape=jax.ShapeDtypeStruct((G, 1), jnp.float32),
    )(zp, W2big, b2big, lin1_w, lin1_b.reshape(1, 128), lin2_w,
      lin2_b.reshape(1, 1))


def kernel(x, W1, b1, W2, b2, W3, b3, W4, b4, conv1_w, conv1_b, conv2_w, conv2_b,
           lin1_w, lin1_b, lin2_w, lin2_b, edge_index, batch):
    src_r = edge_index[0].reshape(NW, NCHUNK, CHUNK)
    dst_r = edge_index[1].reshape(NW, NCHUNK, CHUNK)
    zeros32 = jnp.zeros((NPAD, 32), jnp.float32)
    zeros8 = jnp.zeros((NPAD, 8), jnp.float32)
    zeros16 = jnp.zeros((NPAD, 16), jnp.float32)
    ones_c8 = jnp.ones((CHUNK, 8), jnp.float32)

    deg_p = _deg_kernel(dst_r, ones_c8, zeros8).reshape(NC, NPAD, 8)
    xp = jnp.pad(x, ((0, NPAD - N), (0, 0)))

    hwp1, dinv = _tc_pre(xp, W1, deg_p)
    acc1 = _mp32(hwp1, src_r, dst_r, zeros32).reshape(NC, NPAD, 32)
    h1, hwp2 = _tc_mid(acc1, hwp1, dinv, W2, b1)
    acc2 = _mp32(hwp2, src_r, dst_r, zeros32).reshape(NC, NPAD, 32)
    h2, hwp3 = _tc_mid(acc2, hwp2, dinv, W3, b2)
    acc3 = _mp32(hwp3, src_r, dst_r, zeros32).reshape(NC, NPAD, 32)
    h3, hwp4 = _tc_mid4(acc3, hwp3, dinv, W4, b3)
    acc4 = _mp16(hwp4, src_r, dst_r, zeros16).reshape(NC, NPAD, 16)
    h4 = _tc_post(acc4, hwp4, dinv, b4)

    counts = jnp.bincount(batch, length=G).astype(jnp.int32)
    starts = jnp.cumsum(counts) - counts
    starts_p = jnp.pad(jnp.concatenate(
        [starts, jnp.array([N], jnp.int32)]), (0, 31))
    hcat = jnp.concatenate([h1, h2, h3, h4], axis=-1)
    hcat = jnp.pad(hcat, ((0, 0), (0, 31))).at[N].set(0.0)
    pe, po = _sel_kernel(h4.reshape(NPAD), starts_p, hcat)

    wc1 = jnp.pad(conv1_w[:, 0, :].T, ((0, 128 - 97), (0, 0)))
    blk = conv2_w.transpose(2, 1, 0).reshape(80, 32)
    slabs = jnp.stack([jnp.pad(blk, ((t2 * 16, 160 - t2 * 16), (0, 0)))
                       for t2 in range(11)])
    w2big = slabs.transpose(1, 2, 0).reshape(240, 352)
    b2big = jnp.repeat(conv2_b, 11).reshape(1, 352)

    zp = _tc_headA(pe, po, wc1, conv1_b)
    zpf = zp.reshape(G, 240)
    return _tc_headB(zpf, w2big, b2big, lin1_w, lin1_b, lin2_w, lin2_b)


# speedup vs baseline: 21.9149x; 1.0005x over previous
import functools

import jax
import jax.numpy as jnp
from jax import lax
from jax.experimental import pallas as pl
from jax.experimental.pallas import tpu as pltpu
from jax.experimental.pallas import tpu_sc as plsc

N = 10000
E = 320000
G = 512
K = 30

NC = 2
NS = 16
NW = NC * NS
EPW = E // NW
CHUNK = 80
NCHUNK = EPW // CHUNK
NBUF = 5
NPAD = 10240
RPT = NPAD // NS

_mesh = plsc.VectorSubcoreMesh(core_axis_name="c", subcore_axis_name="s")


@functools.partial(
    pl.kernel,
    out_type=jax.ShapeDtypeStruct((NC * NPAD, 8), jnp.float32),
    mesh=_mesh,
    scratch_types=[
        pltpu.VMEM((NCHUNK, CHUNK), jnp.int32),
        pltpu.VMEM((CHUNK, 8), jnp.float32),
        pltpu.VMEM_SHARED((NPAD, 8), jnp.float32),
        [pltpu.SemaphoreType.DMA] * NBUF,
    ],
    compiler_params=pltpu.CompilerParams(use_tc_tiling_on_sc=False),
)
def _deg_kernel(dst_hbm, ones_hbm, zeros_hbm, out_hbm, dst_v, ones_v, acc_sh, sem):
    c = lax.axis_index("c")
    s = lax.axis_index("s")
    wid = c * NS + s
    r0 = s * RPT

    pltpu.sync_copy(ones_hbm, ones_v)
    pltpu.sync_copy(zeros_hbm.at[pl.ds(r0, RPT)], acc_sh.at[pl.ds(r0, RPT)])
    pltpu.sync_copy(dst_hbm.at[wid], dst_v)
    plsc.subcore_barrier()

    @pl.loop(0, NCHUNK, step=NBUF)
    def _(j0):
        for b in range(NBUF):
            pltpu.async_copy(ones_v, acc_sh.at[dst_v.at[j0 + b]], sem[b],
                             add=True)
        for b in range(NBUF):
            pltpu.make_async_copy(ones_v, acc_sh.at[dst_v.at[j0]],
                                  sem[b]).wait()

    plsc.subcore_barrier()
    pltpu.sync_copy(acc_sh.at[pl.ds(r0, RPT)],
                    out_hbm.at[pl.ds(c * NPAD + r0, RPT)])


def _make_mp(Hc):
    @functools.partial(
        pl.kernel,
        out_type=jax.ShapeDtypeStruct((NC * NPAD, Hc), jnp.float32),
        mesh=_mesh,
        scratch_types=[
            pltpu.VMEM((NCHUNK, CHUNK), jnp.int32),
            pltpu.VMEM((NCHUNK, CHUNK), jnp.int32),
            [pltpu.VMEM((CHUNK, Hc), jnp.float32)] * NBUF,
            pltpu.VMEM_SHARED((NPAD, Hc), jnp.float32),
            [pltpu.SemaphoreType.DMA] * NBUF,
        ],
        compiler_params=pltpu.CompilerParams(use_tc_tiling_on_sc=False),
    )
    def _mp(hw_hbm, src_hbm, dst_hbm, zeros_hbm, out_hbm,
            src_v, dst_v, msg, acc_sh, gsem):
        c = lax.axis_index("c")
        s = lax.axis_index("s")
        wid = c * NS + s
        r0 = s * RPT

        pltpu.sync_copy(zeros_hbm.at[pl.ds(r0, RPT)], acc_sh.at[pl.ds(r0, RPT)])
        pltpu.sync_copy(src_hbm.at[wid], src_v)
        pltpu.sync_copy(dst_hbm.at[wid], dst_v)
        plsc.subcore_barrier()

        for b in range(NBUF - 1):
            pltpu.async_copy(hw_hbm.at[src_v.at[b]], msg[b], gsem[b])

        @pl.loop(0, NCHUNK, step=NBUF)
        def _(j0):
            for b in range(NBUF):
                j = j0 + b
                pltpu.make_async_copy(hw_hbm.at[src_v.at[j]], msg[b],
                                      gsem[b]).wait()
                pltpu.sync_copy(msg[b], acc_sh.at[dst_v.at[j]], add=True)
                jp = j + NBUF - 1
                pb = (b - 1) % NBUF

                @pl.when(jp < NCHUNK)
                def _():
                    pltpu.async_copy(hw_hbm.at[src_v.at[jp]], msg[pb], gsem[pb])

        plsc.subcore_barrier()
        pltpu.sync_copy(acc_sh.at[pl.ds(r0, RPT)],
                        out_hbm.at[pl.ds(c * NPAD + r0, RPT)])

    return _mp


_mp32 = _make_mp(32)
_mp16 = _make_mp(16)


GPT = G // NW


@functools.partial(
    pl.kernel,
    out_type=(jax.ShapeDtypeStruct((G * 15, 128), jnp.float32),
              jax.ShapeDtypeStruct((G * 15, 128), jnp.float32)),
    mesh=_mesh,
    scratch_types=[
        pltpu.VMEM((NPAD,), jnp.float32),
        pltpu.VMEM((G + 32,), jnp.int32),
        pltpu.VMEM((2, 128), jnp.int32),
        pltpu.VMEM((2, 128), jnp.int32),
        pltpu.VMEM((256, 128), jnp.float32),
        pltpu.VMEM((256, 128), jnp.float32),
        [pltpu.SemaphoreType.DMA] * 4,
    ],
    compiler_params=pltpu.CompilerParams(use_tc_tiling_on_sc=False,
                                         needs_layout_passes=False),
)
def _sel_kernel(h4_hbm, starts_hbm, hcat_hbm, pe_hbm, po_hbm,
                val_v, st_v, selE, selO, rowsE, rowsO, sem):
    c = lax.axis_index("c")
    s = lax.axis_index("s")
    wid = c * NS + s
    g0 = wid * GPT

    pltpu.sync_copy(h4_hbm, val_v)
    pltpu.sync_copy(starts_hbm, st_v)
    nsplat = jnp.full((16,), N, jnp.int32)
    for r in range(2):
        for i in range(8):
            selE[r, pl.ds(i * 16, 16)] = nsplat
            selO[r, pl.ds(i * 16, 16)] = nsplat

    lane = lax.iota(jnp.int32, 16)
    sA = st_v[pl.ds(g0, 16)]
    sC = st_v[pl.ds(g0 + 16, 16)]
    sts = [sA[i] for i in range(16)] + [sC[0]]
    for g_local in range(GPT):
        s0 = sts[g_local]
        s1 = sts[g_local + 1]
        a0 = (s0 >> 4) << 4
        nch = (s1 - a0 + 15) >> 4

        @pl.loop(0, nch)
        def _(ic):
            base = a0 + ic * 16
            vi = val_v[pl.ds(base, 16)]
            gidx = base + lane
            ivalid = (gidx >= s0) & (gidx < s1)

            def jstep(jc, rank):
                jbase = a0 + jc * 16
                vj = val_v[pl.ds(jbase, 16)]
                for r in range(16):
                    idxr = (lane + r) & 15
                    vjr = vj[idxr]
                    bidxv = jbase + idxr
                    jvv = (bidxv >= s0) & (bidxv < s1)
                    beats = (vjr > vi) | ((vjr == vi) & (bidxv < gidx))
                    rank = rank + jnp.where(jvv & beats, 1, 0)
                return rank

            rank = pl.loop(0, nch, init_carry=jnp.zeros((16,), jnp.int32))(jstep)

            sel_mask = ivalid & (rank < K)
            te = rank >> 1
            slot = g_local * 15 + te
            row = slot >> 7
            col = slot & 127
            par_odd = (rank & 1) == 1
            plsc.store_scatter(selE, [row, col], gidx, mask=sel_mask & (~par_odd))
            plsc.store_scatter(selO, [row, col], gidx, mask=sel_mask & par_odd)

    for r in range(2):
        pltpu.async_copy(hcat_hbm.at[selE.at[r]],
                         rowsE.at[pl.ds(r * 128, 128)], sem[r])
        pltpu.async_copy(hcat_hbm.at[selO.at[r]],
                         rowsO.at[pl.ds(r * 128, 128)], sem[2 + r])
    for r in range(2):
        pltpu.make_async_copy(hcat_hbm.at[selE.at[r]],
                              rowsE.at[pl.ds(r * 128, 128)], sem[r]).wait()
        pltpu.make_async_copy(hcat_hbm.at[selO.at[r]],
                              rowsO.at[pl.ds(r * 128, 128)], sem[2 + r]).wait()

    pltpu.sync_copy(rowsE.at[pl.ds(0, GPT * 15)],
                    pe_hbm.at[pl.ds(wid * GPT * 15, GPT * 15)])
    pltpu.sync_copy(rowsO.at[pl.ds(0, GPT * 15)],
                    po_hbm.at[pl.ds(wid * GPT * 15, GPT * 15)])


def _pre_body(x_ref, w_ref, degp_ref, hwp_ref, dinv_ref):
    deg = 1.0 + degp_ref[0, :, 0] + degp_ref[1, :, 0]
    dinv = lax.rsqrt(deg)[:, None]
    hw = jnp.dot(x_ref[...], w_ref[...], preferred_element_type=jnp.float32)
    hwp_ref[...] = hw * dinv
    dinv_ref[...] = dinv


def _tc_pre(xp, W1, deg_p):
    return pl.pallas_call(
        _pre_body,
        out_shape=(jax.ShapeDtypeStruct((NPAD, 32), jnp.float32),
                   jax.ShapeDtypeStruct((NPAD, 1), jnp.float32)),
    )(xp, W1, deg_p)


def _mid_body(acc_ref, hw_ref, dinv_ref, w_ref, b_ref, h_ref, hwp_ref):
    agg = acc_ref[0] + acc_ref[1] + hw_ref[...]
    h = jnp.tanh(dinv_ref[...] * agg + b_ref[...])
    h_ref[...] = h
    hwp_ref[...] = jnp.dot(h, w_ref[...],
                           preferred_element_type=jnp.float32) * dinv_ref[...]


def _tc_mid(acc, hwp, dinv, Wn, b):
    return pl.pallas_call(
        _mid_body,
        out_shape=(jax.ShapeDtypeStruct((NPAD, 32), jnp.float32),
                   jax.ShapeDtypeStruct((NPAD, 32), jnp.float32)),
    )(acc, hwp, dinv, Wn, b.reshape(1, 32))


def _mid4_body(acc_ref, hw_ref, dinv_ref, w_ref, b_ref, h_ref, hwp_ref):
    agg = acc_ref[0] + acc_ref[1] + hw_ref[...]
    h = jnp.tanh(dinv_ref[...] * agg + b_ref[...])
    h_ref[...] = h
    hw4 = jnp.dot(h, w_ref[...], preferred_element_type=jnp.float32) * dinv_ref[...]
    hwp_ref[...] = jnp.pad(hw4, ((0, 0), (0, 15)))


def _tc_mid4(acc, hwp, dinv, W4, b):
    return pl.pallas_call(
        _mid4_body,
        out_shape=(jax.ShapeDtypeStruct((NPAD, 32), jnp.float32),
                   jax.ShapeDtypeStruct((NPAD, 16), jnp.float32)),
    )(acc, hwp, dinv, W4, b.reshape(1, 32))


def _post_body(acc_ref, hw_ref, dinv_ref, b_ref, h4_ref):
    agg = acc_ref[0, :, :1] + acc_ref[1, :, :1] + hw_ref[:, :1]
    h4_ref[...] = jnp.tanh(dinv_ref[...] * agg + b_ref[...])


def _tc_post(acc4, hwp4, dinv, b4):
    return pl.pallas_call(
        _post_body,
        out_shape=jax.ShapeDtypeStruct((NPAD, 1), jnp.float32),
    )(acc4, hwp4, dinv, b4.reshape(1, 1))


def _headA_body(pe_ref, po_ref, wc_ref, bc_ref, zp_ref):
    ze = jnp.dot(pe_ref[...], wc_ref[...], preferred_element_type=jnp.float32)
    zo = jnp.dot(po_ref[...], wc_ref[...], preferred_element_type=jnp.float32)
    zp_ref[...] = jnp.maximum(jnp.maximum(ze, zo) + bc_ref[...], 0.0)


def _tc_headA(pe, po, wc, bc):
    return pl.pallas_call(
        _headA_body,
        out_shape=jax.ShapeDtypeStruct((G * 15, 16), jnp.float32),
    )(pe, po, wc, bc.reshape(1, 16))


def _headB_body(zp_ref, w2_ref, b2_ref, l1_ref, lb1_ref, l2_ref, lb2_ref, out_ref):
    z = jnp.maximum(
        jnp.dot(zp_ref[...], w2_ref[...], preferred_element_type=jnp.float32)
        + b2_ref[...], 0.0)
    z = jnp.maximum(
        jnp.dot(z, l1_ref[...], preferred_element_type=jnp.float32)
        + lb1_ref[...], 0.0)
    zo = jnp.dot(z, l2_ref[...], preferred_element_type=jnp.float32) + lb2_ref[...]
    out_ref[...] = jax.nn.sigmoid(zo)


def _tc_headB(zp, W2big, b2big, lin1_w, lin1_b, lin2_w, lin2_b):
    return pl.pallas_call(
        _headB_body,
        out_shape=jax.ShapeDtypeStruct((G, 1), jnp.float32),
    )(zp, W2big, b2big, lin1_w, lin1_b.reshape(1, 128), lin2_w,
      lin2_b.reshape(1, 1))


def kernel(x, W1, b1, W2, b2, W3, b3, W4, b4, conv1_w, conv1_b, conv2_w, conv2_b,
           lin1_w, lin1_b, lin2_w, lin2_b, edge_index, batch):
    src_r = edge_index[0].reshape(NW, NCHUNK, CHUNK)
    dst_r = edge_index[1].reshape(NW, NCHUNK, CHUNK)
    zeros32 = jnp.zeros((NPAD, 32), jnp.float32)
    zeros8 = jnp.zeros((NPAD, 8), jnp.float32)
    zeros16 = jnp.zeros((NPAD, 16), jnp.float32)
    ones_c8 = jnp.ones((CHUNK, 8), jnp.float32)

    deg_p = _deg_kernel(dst_r, ones_c8, zeros8).reshape(NC, NPAD, 8)
    xp = jnp.pad(x, ((0, NPAD - N), (0, 0)))

    hwp1, dinv = _tc_pre(xp, W1, deg_p)
    acc1 = _mp32(hwp1, src_r, dst_r, zeros32).reshape(NC, NPAD, 32)
    h1, hwp2 = _tc_mid(acc1, hwp1, dinv, W2, b1)
    acc2 = _mp32(hwp2, src_r, dst_r, zeros32).reshape(NC, NPAD, 32)
    h2, hwp3 = _tc_mid(acc2, hwp2, dinv, W3, b2)
    acc3 = _mp32(hwp3, src_r, dst_r, zeros32).reshape(NC, NPAD, 32)
    h3, hwp4 = _tc_mid4(acc3, hwp3, dinv, W4, b3)
    acc4 = _mp16(hwp4, src_r, dst_r, zeros16).reshape(NC, NPAD, 16)
    h4 = _tc_post(acc4, hwp4, dinv, b4)

    counts = jnp.bincount(batch, length=G).astype(jnp.int32)
    starts = jnp.cumsum(counts) - counts
    starts_p = jnp.pad(jnp.concatenate(
        [starts, jnp.array([N], jnp.int32)]), (0, 31))
    hcat = jnp.concatenate([h1, h2, h3, h4], axis=-1)
    hcat = jnp.pad(hcat, ((0, 0), (0, 31))).at[N].set(0.0)
    pe, po = _sel_kernel(h4.reshape(NPAD), starts_p, hcat)

    wc1 = jnp.pad(conv1_w[:, 0, :].T, ((0, 128 - 97), (0, 0)))
    blk = conv2_w.transpose(2, 1, 0).reshape(80, 32)
    slabs = jnp.stack([jnp.pad(blk, ((t2 * 16, 160 - t2 * 16), (0, 0)))
                       for t2 in range(11)])
    w2big = slabs.transpose(1, 2, 0).reshape(240, 352)
    b2big = jnp.repeat(conv2_b, 11).reshape(1, 352)

    zp = _tc_headA(pe, po, wc1, conv1_b)
    zpf = zp.reshape(G, 240)
    return _tc_headB(zpf, w2big, b2big, lin1_w, lin1_b, lin2_w, lin2_b)


# speedup vs baseline: 33.0995x; 1.5104x over previous
import functools

import jax
import jax.numpy as jnp
from jax import lax
from jax.experimental import pallas as pl
from jax.experimental.pallas import tpu as pltpu
from jax.experimental.pallas import tpu_sc as plsc

N = 10000
E = 320000
G = 512
K = 30

NC = 2
NS = 16
NW = NC * NS
EPW = E // NW
CHUNK = 80
NCHUNK = EPW // CHUNK
NBUF = 5
NPAD = 10240
RPT = NPAD // NS

_mesh = plsc.VectorSubcoreMesh(core_axis_name="c", subcore_axis_name="s")


@functools.partial(
    pl.kernel,
    out_type=jax.ShapeDtypeStruct((NC * NPAD, 8), jnp.float32),
    mesh=_mesh,
    scratch_types=[
        pltpu.VMEM((NCHUNK, CHUNK), jnp.int32),
        pltpu.VMEM((CHUNK, 8), jnp.float32),
        pltpu.VMEM_SHARED((NPAD, 8), jnp.float32),
        [pltpu.SemaphoreType.DMA] * NBUF,
    ],
    compiler_params=pltpu.CompilerParams(use_tc_tiling_on_sc=False),
)
def _deg_kernel(dst_hbm, ones_hbm, zeros_hbm, out_hbm, dst_v, ones_v, acc_sh, sem):
    c = lax.axis_index("c")
    s = lax.axis_index("s")
    wid = c * NS + s
    r0 = s * RPT

    pltpu.sync_copy(ones_hbm, ones_v)
    pltpu.sync_copy(zeros_hbm.at[pl.ds(r0, RPT)], acc_sh.at[pl.ds(r0, RPT)])
    pltpu.sync_copy(dst_hbm.at[wid], dst_v)
    plsc.subcore_barrier()

    @pl.loop(0, NCHUNK, step=NBUF)
    def _(j0):
        for b in range(NBUF):
            pltpu.async_copy(ones_v, acc_sh.at[dst_v.at[j0 + b]], sem[b],
                             add=True)
        for b in range(NBUF):
            pltpu.make_async_copy(ones_v, acc_sh.at[dst_v.at[j0]],
                                  sem[b]).wait()

    plsc.subcore_barrier()
    pltpu.sync_copy(acc_sh.at[pl.ds(r0, RPT)],
                    out_hbm.at[pl.ds(c * NPAD + r0, RPT)])


def _make_mp(Hc):
    @functools.partial(
        pl.kernel,
        out_type=jax.ShapeDtypeStruct((NC * NPAD, Hc), jnp.float32),
        mesh=_mesh,
        scratch_types=[
            pltpu.VMEM((NCHUNK, CHUNK), jnp.int32),
            pltpu.VMEM((NCHUNK, CHUNK), jnp.int32),
            [pltpu.VMEM((CHUNK, Hc), jnp.float32)] * NBUF,
            pltpu.VMEM_SHARED((NPAD, Hc), jnp.float32),
            [pltpu.SemaphoreType.DMA] * NBUF,
        ],
        compiler_params=pltpu.CompilerParams(use_tc_tiling_on_sc=False),
    )
    def _mp(hw_hbm, src_hbm, dst_hbm, zeros_hbm, out_hbm,
            src_v, dst_v, msg, acc_sh, gsem):
        c = lax.axis_index("c")
        s = lax.axis_index("s")
        wid = c * NS + s
        r0 = s * RPT

        pltpu.sync_copy(zeros_hbm.at[pl.ds(r0, RPT)], acc_sh.at[pl.ds(r0, RPT)])
        pltpu.sync_copy(src_hbm.at[wid], src_v)
        pltpu.sync_copy(dst_hbm.at[wid], dst_v)
        plsc.subcore_barrier()

        for b in range(NBUF - 1):
            pltpu.async_copy(hw_hbm.at[src_v.at[b]], msg[b], gsem[b])

        @pl.loop(0, NCHUNK, step=NBUF)
        def _(j0):
            for b in range(NBUF):
                j = j0 + b
                pltpu.make_async_copy(hw_hbm.at[src_v.at[j]], msg[b],
                                      gsem[b]).wait()
                pltpu.sync_copy(msg[b], acc_sh.at[dst_v.at[j]], add=True)
                jp = j + NBUF - 1
                pb = (b - 1) % NBUF

                @pl.when(jp < NCHUNK)
                def _():
                    pltpu.async_copy(hw_hbm.at[src_v.at[jp]], msg[pb], gsem[pb])

        plsc.subcore_barrier()
        pltpu.sync_copy(acc_sh.at[pl.ds(r0, RPT)],
                        out_hbm.at[pl.ds(c * NPAD + r0, RPT)])

    return _mp


_mp32 = _make_mp(32)
_mp16 = _make_mp(16)


GPT = G // NW


@functools.partial(
    pl.kernel,
    out_type=(jax.ShapeDtypeStruct((G * 15, 128), jnp.float32),
              jax.ShapeDtypeStruct((G * 15, 128), jnp.float32)),
    mesh=_mesh,
    scratch_types=[
        pltpu.VMEM((NPAD,), jnp.float32),
        pltpu.VMEM((G + 32,), jnp.int32),
        pltpu.VMEM((2, 128), jnp.int32),
        pltpu.VMEM((2, 128), jnp.int32),
        pltpu.VMEM((256, 128), jnp.float32),
        pltpu.VMEM((256, 128), jnp.float32),
        [pltpu.SemaphoreType.DMA] * 4,
    ],
    compiler_params=pltpu.CompilerParams(use_tc_tiling_on_sc=False,
                                         needs_layout_passes=False),
)
def _sel_kernel(h4_hbm, starts_hbm, hcat_hbm, pe_hbm, po_hbm,
                val_v, st_v, selE, selO, rowsE, rowsO, sem):
    c = lax.axis_index("c")
    s = lax.axis_index("s")
    wid = c * NS + s
    g0 = wid * GPT

    pltpu.sync_copy(h4_hbm, val_v)
    pltpu.sync_copy(starts_hbm, st_v)
    lane0 = lax.iota(jnp.int32, 16)
    for r in range(2):
        for i in range(8):
            pad_idx = N + ((r * 128 + i * 16) + lane0 * 8) % 224
            selE[r, pl.ds(i * 16, 16)] = pad_idx
            selO[r, pl.ds(i * 16, 16)] = pad_idx + 8

    lane = lax.iota(jnp.int32, 16)
    sA = st_v[pl.ds(g0, 16)]
    sC = st_v[pl.ds(g0 + 16, 16)]
    sts = [sA[i] for i in range(16)] + [sC[0]]
    for g_local in range(GPT):
        s0 = sts[g_local]
        s1 = sts[g_local + 1]
        a0 = (s0 >> 4) << 4
        nch = (s1 - a0 + 15) >> 4

        @pl.loop(0, nch)
        def _(ic):
            base = a0 + ic * 16
            vi = val_v[pl.ds(base, 16)]
            gidx = base + lane
            ivalid = (gidx >= s0) & (gidx < s1)

            def jstep(jc, rank):
                jbase = a0 + jc * 16
                vj = val_v[pl.ds(jbase, 16)]
                for r in range(16):
                    idxr = (lane + r) & 15
                    vjr = vj[idxr]
                    bidxv = jbase + idxr
                    jvv = (bidxv >= s0) & (bidxv < s1)
                    beats = (vjr > vi) | ((vjr == vi) & (bidxv < gidx))
                    rank = rank + jnp.where(jvv & beats, 1, 0)
                return rank

            rank = pl.loop(0, nch, init_carry=jnp.zeros((16,), jnp.int32))(jstep)

            sel_mask = ivalid & (rank < K)
            te = rank >> 1
            slot = g_local * 15 + te
            row = slot >> 7
            col = slot & 127
            par_odd = (rank & 1) == 1
            plsc.store_scatter(selE, [row, col], gidx, mask=sel_mask & (~par_odd))
            plsc.store_scatter(selO, [row, col], gidx, mask=sel_mask & par_odd)

    for r in range(2):
        pltpu.async_copy(hcat_hbm.at[selE.at[r]],
                         rowsE.at[pl.ds(r * 128, 128)], sem[r])
        pltpu.async_copy(hcat_hbm.at[selO.at[r]],
                         rowsO.at[pl.ds(r * 128, 128)], sem[2 + r])
    for r in range(2):
        pltpu.make_async_copy(hcat_hbm.at[selE.at[r]],
                              rowsE.at[pl.ds(r * 128, 128)], sem[r]).wait()
        pltpu.make_async_copy(hcat_hbm.at[selO.at[r]],
                              rowsO.at[pl.ds(r * 128, 128)], sem[2 + r]).wait()

    pltpu.sync_copy(rowsE.at[pl.ds(0, GPT * 15)],
                    pe_hbm.at[pl.ds(wid * GPT * 15, GPT * 15)])
    pltpu.sync_copy(rowsO.at[pl.ds(0, GPT * 15)],
                    po_hbm.at[pl.ds(wid * GPT * 15, GPT * 15)])


def _pre_body(x_ref, w_ref, degp_ref, hwp_ref, dinv_ref):
    deg = 1.0 + degp_ref[0, :, 0] + degp_ref[1, :, 0]
    dinv = lax.rsqrt(deg)[:, None]
    hw = jnp.dot(x_ref[...], w_ref[...], preferred_element_type=jnp.float32)
    hwp_ref[...] = hw * dinv
    dinv_ref[...] = dinv


def _tc_pre(xp, W1, deg_p):
    return pl.pallas_call(
        _pre_body,
        out_shape=(jax.ShapeDtypeStruct((NPAD, 32), jnp.float32),
                   jax.ShapeDtypeStruct((NPAD, 1), jnp.float32)),
    )(xp, W1, deg_p)


def _mid_body(acc_ref, hw_ref, dinv_ref, w_ref, b_ref, h_ref, hwp_ref):
    agg = acc_ref[0] + acc_ref[1] + hw_ref[...]
    h = jnp.tanh(dinv_ref[...] * agg + b_ref[...])
    h_ref[...] = h
    hwp_ref[...] = jnp.dot(h, w_ref[...],
                           preferred_element_type=jnp.float32) * dinv_ref[...]


def _tc_mid(acc, hwp, dinv, Wn, b):
    return pl.pallas_call(
        _mid_body,
        out_shape=(jax.ShapeDtypeStruct((NPAD, 32), jnp.float32),
                   jax.ShapeDtypeStruct((NPAD, 32), jnp.float32)),
    )(acc, hwp, dinv, Wn, b.reshape(1, 32))


def _mid4_body(acc_ref, hw_ref, dinv_ref, w_ref, b_ref, h_ref, hwp_ref):
    agg = acc_ref[0] + acc_ref[1] + hw_ref[...]
    h = jnp.tanh(dinv_ref[...] * agg + b_ref[...])
    h_ref[...] = h
    hw4 = jnp.dot(h, w_ref[...], preferred_element_type=jnp.float32) * dinv_ref[...]
    hwp_ref[...] = jnp.pad(hw4, ((0, 0), (0, 15)))


def _tc_mid4(acc, hwp, dinv, W4, b):
    return pl.pallas_call(
        _mid4_body,
        out_shape=(jax.ShapeDtypeStruct((NPAD, 32), jnp.float32),
                   jax.ShapeDtypeStruct((NPAD, 16), jnp.float32)),
    )(acc, hwp, dinv, W4, b.reshape(1, 32))


def _post_body(acc_ref, hw_ref, dinv_ref, b_ref, h4_ref):
    agg = acc_ref[0, :, :1] + acc_ref[1, :, :1] + hw_ref[:, :1]
    h4_ref[...] = jnp.tanh(dinv_ref[...] * agg + b_ref[...])


def _tc_post(acc4, hwp4, dinv, b4):
    return pl.pallas_call(
        _post_body,
        out_shape=jax.ShapeDtypeStruct((NPAD, 1), jnp.float32),
    )(acc4, hwp4, dinv, b4.reshape(1, 1))


def _headA_body(pe_ref, po_ref, wc_ref, bc_ref, zp_ref):
    ze = jnp.dot(pe_ref[...], wc_ref[...], preferred_element_type=jnp.float32)
    zo = jnp.dot(po_ref[...], wc_ref[...], preferred_element_type=jnp.float32)
    zp_ref[...] = jnp.maximum(jnp.maximum(ze, zo) + bc_ref[...], 0.0)


def _tc_headA(pe, po, wc, bc):
    return pl.pallas_call(
        _headA_body,
        out_shape=jax.ShapeDtypeStruct((G * 15, 16), jnp.float32),
    )(pe, po, wc, bc.reshape(1, 16))


def _headB_body(zp_ref, w2_ref, b2_ref, l1_ref, lb1_ref, l2_ref, lb2_ref, out_ref):
    z = jnp.maximum(
        jnp.dot(zp_ref[...], w2_ref[...], preferred_element_type=jnp.float32)
        + b2_ref[...], 0.0)
    z = jnp.maximum(
        jnp.dot(z, l1_ref[...], preferred_element_type=jnp.float32)
        + lb1_ref[...], 0.0)
    zo = jnp.dot(z, l2_ref[...], preferred_element_type=jnp.float32) + lb2_ref[...]
    out_ref[...] = jax.nn.sigmoid(zo)


def _tc_headB(zp, W2big, b2big, lin1_w, lin1_b, lin2_w, lin2_b):
    return pl.pallas_call(
        _headB_body,
        out_shape=jax.ShapeDtypeStruct((G, 1), jnp.float32),
    )(zp, W2big, b2big, lin1_w, lin1_b.reshape(1, 128), lin2_w,
      lin2_b.reshape(1, 1))


def kernel(x, W1, b1, W2, b2, W3, b3, W4, b4, conv1_w, conv1_b, conv2_w, conv2_b,
           lin1_w, lin1_b, lin2_w, lin2_b, edge_index, batch):
    src_r = edge_index[0].reshape(NW, NCHUNK, CHUNK)
    dst_r = edge_index[1].reshape(NW, NCHUNK, CHUNK)
    zeros32 = jnp.zeros((NPAD, 32), jnp.float32)
    zeros8 = jnp.zeros((NPAD, 8), jnp.float32)
    zeros16 = jnp.zeros((NPAD, 16), jnp.float32)
    ones_c8 = jnp.ones((CHUNK, 8), jnp.float32)

    deg_p = _deg_kernel(dst_r, ones_c8, zeros8).reshape(NC, NPAD, 8)
    xp = jnp.pad(x, ((0, NPAD - N), (0, 0)))

    hwp1, dinv = _tc_pre(xp, W1, deg_p)
    acc1 = _mp32(hwp1, src_r, dst_r, zeros32).reshape(NC, NPAD, 32)
    h1, hwp2 = _tc_mid(acc1, hwp1, dinv, W2, b1)
    acc2 = _mp32(hwp2, src_r, dst_r, zeros32).reshape(NC, NPAD, 32)
    h2, hwp3 = _tc_mid(acc2, hwp2, dinv, W3, b2)
    acc3 = _mp32(hwp3, src_r, dst_r, zeros32).reshape(NC, NPAD, 32)
    h3, hwp4 = _tc_mid4(acc3, hwp3, dinv, W4, b3)
    acc4 = _mp16(hwp4, src_r, dst_r, zeros16).reshape(NC, NPAD, 16)
    h4 = _tc_post(acc4, hwp4, dinv, b4)

    counts = jnp.bincount(batch, length=G).astype(jnp.int32)
    starts = jnp.cumsum(counts) - counts
    starts_p = jnp.pad(jnp.concatenate(
        [starts, jnp.array([N], jnp.int32)]), (0, 31))
    hfeat = jnp.concatenate([h1[:N], h2[:N], h3[:N], h4[:N]], axis=-1)
    hcat = jnp.zeros((NPAD, 128), jnp.float32).at[:N, :97].set(hfeat)
    pe, po = _sel_kernel(h4.reshape(NPAD), starts_p, hcat)

    wc1 = jnp.pad(conv1_w[:, 0, :].T, ((0, 128 - 97), (0, 0)))
    blk = conv2_w.transpose(2, 1, 0).reshape(80, 32)
    slabs = jnp.stack([jnp.pad(blk, ((t2 * 16, 160 - t2 * 16), (0, 0)))
                       for t2 in range(11)])
    w2big = slabs.transpose(1, 2, 0).reshape(240, 352)
    b2big = jnp.repeat(conv2_b, 11).reshape(1, 352)

    zp = _tc_headA(pe, po, wc1, conv1_b)
    zpf = zp.reshape(G, 240)
    return _tc_headB(zpf, w2big, b2big, lin1_w, lin1_b, lin2_w, lin2_b)


# speedup vs baseline: 33.4879x; 1.0117x over previous
import functools

import jax
import jax.numpy as jnp
from jax import lax
from jax.experimental import pallas as pl
from jax.experimental.pallas import tpu as pltpu
from jax.experimental.pallas import tpu_sc as plsc

N = 10000
E = 320000
G = 512
K = 30

NC = 2
NS = 16
NW = NC * NS
EPW = E // NW
CHUNK = 80
NCHUNK = EPW // CHUNK
NBUF = 5
NPAD = 10240
RPT = NPAD // NS

_mesh = plsc.VectorSubcoreMesh(core_axis_name="c", subcore_axis_name="s")


@functools.partial(
    pl.kernel,
    out_type=(jax.ShapeDtypeStruct((NC * NPAD, 8), jnp.float32),
              jax.ShapeDtypeStruct((NC * 640, 8), jnp.float32)),
    mesh=_mesh,
    scratch_types=[
        pltpu.VMEM((NCHUNK, CHUNK), jnp.int32),
        pltpu.VMEM((4, CHUNK), jnp.int32),
        pltpu.VMEM((CHUNK, 8), jnp.float32),
        pltpu.VMEM_SHARED((NPAD, 8), jnp.float32),
        pltpu.VMEM_SHARED((640, 8), jnp.float32),
        [pltpu.SemaphoreType.DMA] * NBUF,
    ],
    compiler_params=pltpu.CompilerParams(use_tc_tiling_on_sc=False),
)
def _deg_kernel(dst_hbm, batch_hbm, ones_hbm, zeros_hbm, out_hbm, cnt_hbm,
                dst_v, bat_v, ones_v, acc_sh, cnt_sh, sem):
    c = lax.axis_index("c")
    s = lax.axis_index("s")
    wid = c * NS + s
    r0 = s * RPT

    pltpu.sync_copy(ones_hbm, ones_v)
    pltpu.sync_copy(zeros_hbm.at[pl.ds(r0, RPT)], acc_sh.at[pl.ds(r0, RPT)])
    pltpu.sync_copy(zeros_hbm.at[pl.ds(s * 40, 40)], cnt_sh.at[pl.ds(s * 40, 40)])
    pltpu.sync_copy(dst_hbm.at[wid], dst_v)
    pltpu.sync_copy(batch_hbm.at[wid], bat_v)
    plsc.subcore_barrier()

    for b in range(4):
        pltpu.async_copy(ones_v, cnt_sh.at[bat_v.at[b]], sem[b], add=True)

    @pl.loop(0, NCHUNK, step=NBUF)
    def _(j0):
        for b in range(NBUF):
            pltpu.async_copy(ones_v, acc_sh.at[dst_v.at[j0 + b]], sem[b],
                             add=True)
        for b in range(NBUF):
            pltpu.make_async_copy(ones_v, acc_sh.at[dst_v.at[j0]],
                                  sem[b]).wait()

    for b in range(4):
        pltpu.make_async_copy(ones_v, cnt_sh.at[bat_v.at[b]], sem[b]).wait()

    plsc.subcore_barrier()
    pltpu.sync_copy(acc_sh.at[pl.ds(r0, RPT)],
                    out_hbm.at[pl.ds(c * NPAD + r0, RPT)])
    pltpu.sync_copy(cnt_sh.at[pl.ds(s * 40, 40)],
                    cnt_hbm.at[pl.ds(c * 640 + s * 40, 40)])


def _make_mp(Hc):
    @functools.partial(
        pl.kernel,
        out_type=jax.ShapeDtypeStruct((NC * NPAD, Hc), jnp.float32),
        mesh=_mesh,
        scratch_types=[
            pltpu.VMEM((NCHUNK, CHUNK), jnp.int32),
            pltpu.VMEM((NCHUNK, CHUNK), jnp.int32),
            [pltpu.VMEM((CHUNK, Hc), jnp.float32)] * NBUF,
            pltpu.VMEM_SHARED((NPAD, Hc), jnp.float32),
            [pltpu.SemaphoreType.DMA] * NBUF,
        ],
        compiler_params=pltpu.CompilerParams(use_tc_tiling_on_sc=False),
    )
    def _mp(hw_hbm, src_hbm, dst_hbm, zeros_hbm, out_hbm,
            src_v, dst_v, msg, acc_sh, gsem):
        c = lax.axis_index("c")
        s = lax.axis_index("s")
        wid = c * NS + s
        r0 = s * RPT

        pltpu.sync_copy(zeros_hbm.at[pl.ds(r0, RPT)], acc_sh.at[pl.ds(r0, RPT)])
        pltpu.sync_copy(src_hbm.at[wid], src_v)
        pltpu.sync_copy(dst_hbm.at[wid], dst_v)
        plsc.subcore_barrier()

        for b in range(NBUF - 1):
            pltpu.async_copy(hw_hbm.at[src_v.at[b]], msg[b], gsem[b])

        @pl.loop(0, NCHUNK, step=NBUF)
        def _(j0):
            for b in range(NBUF):
                j = j0 + b
                pltpu.make_async_copy(hw_hbm.at[src_v.at[j]], msg[b],
                                      gsem[b]).wait()
                pltpu.sync_copy(msg[b], acc_sh.at[dst_v.at[j]], add=True)
                jp = j + NBUF - 1
                pb = (b - 1) % NBUF

                @pl.when(jp < NCHUNK)
                def _():
                    pltpu.async_copy(hw_hbm.at[src_v.at[jp]], msg[pb], gsem[pb])

        plsc.subcore_barrier()
        pltpu.sync_copy(acc_sh.at[pl.ds(r0, RPT)],
                        out_hbm.at[pl.ds(c * NPAD + r0, RPT)])

    return _mp


_mp32 = _make_mp(32)
_mp16 = _make_mp(16)


GPT = G // NW


@functools.partial(
    pl.kernel,
    out_type=(jax.ShapeDtypeStruct((G * 15, 128), jnp.float32),
              jax.ShapeDtypeStruct((G * 15, 128), jnp.float32)),
    mesh=_mesh,
    scratch_types=[
        pltpu.VMEM((NPAD,), jnp.float32),
        pltpu.VMEM((G + 32,), jnp.int32),
        pltpu.VMEM((2, 128), jnp.int32),
        pltpu.VMEM((2, 128), jnp.int32),
        pltpu.VMEM((256, 128), jnp.float32),
        pltpu.VMEM((256, 128), jnp.float32),
        [pltpu.SemaphoreType.DMA] * 4,
    ],
    compiler_params=pltpu.CompilerParams(use_tc_tiling_on_sc=False,
                                         needs_layout_passes=False),
)
def _sel_kernel(h4_hbm, starts_hbm, hcat_hbm, pe_hbm, po_hbm,
                val_v, st_v, selE, selO, rowsE, rowsO, sem):
    c = lax.axis_index("c")
    s = lax.axis_index("s")
    wid = c * NS + s
    g0 = wid * GPT

    pltpu.sync_copy(h4_hbm, val_v)
    pltpu.sync_copy(starts_hbm, st_v)
    lane0 = lax.iota(jnp.int32, 16)
    for r in range(2):
        for i in range(8):
            pad_idx = N + ((r * 128 + i * 16) + lane0 * 8) % 224
            selE[r, pl.ds(i * 16, 16)] = pad_idx
            selO[r, pl.ds(i * 16, 16)] = pad_idx + 8

    lane = lax.iota(jnp.int32, 16)
    sA = st_v[pl.ds(g0, 16)]
    sC = st_v[pl.ds(g0 + 16, 16)]
    sts = [sA[i] for i in range(16)] + [sC[0]]
    for g_local in range(GPT):
        s0 = sts[g_local]
        s1 = sts[g_local + 1]
        a0 = (s0 >> 4) << 4
        nch = (s1 - a0 + 15) >> 4

        @pl.loop(0, nch)
        def _(ic):
            base = a0 + ic * 16
            vi = val_v[pl.ds(base, 16)]
            gidx = base + lane
            ivalid = (gidx >= s0) & (gidx < s1)

            def jstep(jc, rank):
                jbase = a0 + jc * 16
                vj = val_v[pl.ds(jbase, 16)]
                for r in range(16):
                    idxr = (lane + r) & 15
                    vjr = vj[idxr]
                    bidxv = jbase + idxr
                    jvv = (bidxv >= s0) & (bidxv < s1)
                    beats = (vjr > vi) | ((vjr == vi) & (bidxv < gidx))
                    rank = rank + jnp.where(jvv & beats, 1, 0)
                return rank

            rank = pl.loop(0, nch, init_carry=jnp.zeros((16,), jnp.int32))(jstep)

            sel_mask = ivalid & (rank < K)
            te = rank >> 1
            slot = g_local * 15 + te
            row = slot >> 7
            col = slot & 127
            par_odd = (rank & 1) == 1
            plsc.store_scatter(selE, [row, col], gidx, mask=sel_mask & (~par_odd))
            plsc.store_scatter(selO, [row, col], gidx, mask=sel_mask & par_odd)

    for r in range(2):
        pltpu.async_copy(hcat_hbm.at[selE.at[r]],
                         rowsE.at[pl.ds(r * 128, 128)], sem[r])
        pltpu.async_copy(hcat_hbm.at[selO.at[r]],
                         rowsO.at[pl.ds(r * 128, 128)], sem[2 + r])
    for r in range(2):
        pltpu.make_async_copy(hcat_hbm.at[selE.at[r]],
                              rowsE.at[pl.ds(r * 128, 128)], sem[r]).wait()
        pltpu.make_async_copy(hcat_hbm.at[selO.at[r]],
                              rowsO.at[pl.ds(r * 128, 128)], sem[2 + r]).wait()

    pltpu.sync_copy(rowsE.at[pl.ds(0, GPT * 15)],
                    pe_hbm.at[pl.ds(wid * GPT * 15, GPT * 15)])
    pltpu.sync_copy(rowsO.at[pl.ds(0, GPT * 15)],
                    po_hbm.at[pl.ds(wid * GPT * 15, GPT * 15)])


def _pre_body(x_ref, w_ref, degp_ref, hwp_ref, dinv_ref):
    deg = 1.0 + degp_ref[0, :, 0] + degp_ref[1, :, 0]
    dinv = lax.rsqrt(deg)[:, None]
    hw = jnp.dot(x_ref[...], w_ref[...], preferred_element_type=jnp.float32)
    hwp_ref[...] = hw * dinv
    dinv_ref[...] = dinv


def _tc_pre(xp, W1, deg_p):
    return pl.pallas_call(
        _pre_body,
        out_shape=(jax.ShapeDtypeStruct((NPAD, 32), jnp.float32),
                   jax.ShapeDtypeStruct((NPAD, 1), jnp.float32)),
    )(xp, W1, deg_p)


def _mid_body(acc_ref, hw_ref, dinv_ref, w_ref, b_ref, h_ref, hwp_ref):
    agg = acc_ref[0] + acc_ref[1] + hw_ref[...]
    h = jnp.tanh(dinv_ref[...] * agg + b_ref[...])
    h_ref[...] = h
    hwp_ref[...] = jnp.dot(h, w_ref[...],
                           preferred_element_type=jnp.float32) * dinv_ref[...]


def _tc_mid(acc, hwp, dinv, Wn, b):
    return pl.pallas_call(
        _mid_body,
        out_shape=(jax.ShapeDtypeStruct((NPAD, 32), jnp.float32),
                   jax.ShapeDtypeStruct((NPAD, 32), jnp.float32)),
    )(acc, hwp, dinv, Wn, b.reshape(1, 32))


def _mid4_body(acc_ref, hw_ref, dinv_ref, w_ref, b_ref, h_ref, hwp_ref):
    agg = acc_ref[0] + acc_ref[1] + hw_ref[...]
    h = jnp.tanh(dinv_ref[...] * agg + b_ref[...])
    h_ref[...] = h
    hw4 = jnp.dot(h, w_ref[...], preferred_element_type=jnp.float32) * dinv_ref[...]
    hwp_ref[...] = jnp.pad(hw4, ((0, 0), (0, 15)))


def _tc_mid4(acc, hwp, dinv, W4, b):
    return pl.pallas_call(
        _mid4_body,
        out_shape=(jax.ShapeDtypeStruct((NPAD, 32), jnp.float32),
                   jax.ShapeDtypeStruct((NPAD, 16), jnp.float32)),
    )(acc, hwp, dinv, W4, b.reshape(1, 32))


def _post_body(acc_ref, hw_ref, dinv_ref, b_ref, h4_ref):
    agg = acc_ref[0, :, :1] + acc_ref[1, :, :1] + hw_ref[:, :1]
    h4_ref[...] = jnp.tanh(dinv_ref[...] * agg + b_ref[...])


def _tc_post(acc4, hwp4, dinv, b4):
    return pl.pallas_call(
        _post_body,
        out_shape=jax.ShapeDtypeStruct((NPAD, 1), jnp.float32),
    )(acc4, hwp4, dinv, b4.reshape(1, 1))


def _headA_body(pe_ref, po_ref, wc_ref, bc_ref, zp_ref):
    ze = jnp.dot(pe_ref[...], wc_ref[...], preferred_element_type=jnp.float32)
    zo = jnp.dot(po_ref[...], wc_ref[...], preferred_element_type=jnp.float32)
    zp_ref[...] = jnp.maximum(jnp.maximum(ze, zo) + bc_ref[...], 0.0)


def _tc_headA(pe, po, wc, bc):
    return pl.pallas_call(
        _headA_body,
        out_shape=jax.ShapeDtypeStruct((G * 15, 16), jnp.float32),
    )(pe, po, wc, bc.reshape(1, 16))


def _headB_body(zp_ref, w2_ref, b2_ref, l1_ref, lb1_ref, l2_ref, lb2_ref, out_ref):
    z = jnp.maximum(
        jnp.dot(zp_ref[...], w2_ref[...], preferred_element_type=jnp.float32)
        + b2_ref[...], 0.0)
    z = jnp.maximum(
        jnp.dot(z, l1_ref[...], preferred_element_type=jnp.float32)
        + lb1_ref[...], 0.0)
    zo = jnp.dot(z, l2_ref[...], preferred_element_type=jnp.float32) + lb2_ref[...]
    out_ref[...] = jax.nn.sigmoid(zo)


def _tc_headB(zp, W2big, b2big, lin1_w, lin1_b, lin2_w, lin2_b):
    return pl.pallas_call(
        _headB_body,
        out_shape=jax.ShapeDtypeStruct((G, 1), jnp.float32),
    )(zp, W2big, b2big, lin1_w, lin1_b.reshape(1, 128), lin2_w,
      lin2_b.reshape(1, 1))


def kernel(x, W1, b1, W2, b2, W3, b3, W4, b4, conv1_w, conv1_b, conv2_w, conv2_b,
           lin1_w, lin1_b, lin2_w, lin2_b, edge_index, batch):
    src_r = edge_index[0].reshape(NW, NCHUNK, CHUNK)
    dst_r = edge_index[1].reshape(NW, NCHUNK, CHUNK)
    zeros32 = jnp.zeros((NPAD, 32), jnp.float32)
    zeros8 = jnp.zeros((NPAD, 8), jnp.float32)
    zeros16 = jnp.zeros((NPAD, 16), jnp.float32)
    ones_c8 = jnp.ones((CHUNK, 8), jnp.float32)

    batch_r = jnp.pad(batch, (0, NPAD - N), constant_values=560).reshape(
        NW, 4, CHUNK)
    deg_p, cnt_p = _deg_kernel(dst_r, batch_r, ones_c8, zeros8)
    deg_p = deg_p.reshape(NC, NPAD, 8)
    cnt_p = cnt_p.reshape(NC, 640, 8)
    xp = jnp.pad(x, ((0, NPAD - N), (0, 0)))

    hwp1, dinv = _tc_pre(xp, W1, deg_p)
    acc1 = _mp32(hwp1, src_r, dst_r, zeros32).reshape(NC, NPAD, 32)
    h1, hwp2 = _tc_mid(acc1, hwp1, dinv, W2, b1)
    acc2 = _mp32(hwp2, src_r, dst_r, zeros32).reshape(NC, NPAD, 32)
    h2, hwp3 = _tc_mid(acc2, hwp2, dinv, W3, b2)
    acc3 = _mp32(hwp3, src_r, dst_r, zeros32).reshape(NC, NPAD, 32)
    h3, hwp4 = _tc_mid4(acc3, hwp3, dinv, W4, b3)
    acc4 = _mp16(hwp4, src_r, dst_r, zeros16).reshape(NC, NPAD, 16)
    h4 = _tc_post(acc4, hwp4, dinv, b4)

    counts = (cnt_p[0, :G, 0] + cnt_p[1, :G, 0]).astype(jnp.int32)
    starts = jnp.cumsum(counts) - counts
    starts_p = jnp.pad(jnp.concatenate(
        [starts, jnp.array([N], jnp.int32)]), (0, 31))
    hfeat = jnp.concatenate([h1[:N], h2[:N], h3[:N], h4[:N]], axis=-1)
    hcat = jnp.zeros((NPAD, 128), jnp.float32).at[:N, :97].set(hfeat)
    pe, po = _sel_kernel(h4.reshape(NPAD), starts_p, hcat)

    wc1 = jnp.pad(conv1_w[:, 0, :].T, ((0, 128 - 97), (0, 0)))
    blk = conv2_w.transpose(2, 1, 0).reshape(80, 32)
    slabs = jnp.stack([jnp.pad(blk, ((t2 * 16, 160 - t2 * 16), (0, 0)))
                       for t2 in range(11)])
    w2big = slabs.transpose(1, 2, 0).reshape(240, 352)
    b2big = jnp.repeat(conv2_b, 11).reshape(1, 352)

    zp = _tc_headA(pe, po, wc1, conv1_b)
    zpf = zp.reshape(G, 240)
    return _tc_headB(zpf, w2big, b2big, lin1_w, lin1_b, lin2_w, lin2_b)
